# baseline (speedup 1.0000x reference)
import functools

import jax
import jax.numpy as jnp
from jax import lax
from jax.experimental import pallas as pl
from jax.experimental.pallas import tpu as pltpu
from jax.experimental.pallas import tpu_sc as plsc

HEAD_SIZE = 128
HALF = HEAD_SIZE // 2
BATCH = 4
SEQ = 8192
N = BATCH * SEQ
CHUNK = 128
N_ROWS = N // CHUNK


@functools.cache
def _build_sc_kernel():
    info = plsc.get_sparse_core_info()
    nc, ns = info.num_cores, info.num_subcores
    nw = nc * ns
    rows_w = N_ROWS // nw

    mesh = plsc.VectorSubcoreMesh(core_axis_name="c", subcore_axis_name="s")

    @functools.partial(
        pl.kernel,
        mesh=mesh,
        out_type=(
            jax.ShapeDtypeStruct((N // 2, HEAD_SIZE), jnp.float32),
            jax.ShapeDtypeStruct((N // 2, HEAD_SIZE), jnp.float32),
        ),
        scratch_types=[
            pltpu.VMEM((rows_w, CHUNK), jnp.int32),
            pltpu.VMEM((2, CHUNK, HEAD_SIZE), jnp.float32),
            pltpu.VMEM((2, CHUNK // 2, HEAD_SIZE), jnp.float32),
            pltpu.VMEM((2, CHUNK // 2, HEAD_SIZE), jnp.float32),
            pltpu.SemaphoreType.DMA,
            pltpu.SemaphoreType.DMA,
            pltpu.SemaphoreType.DMA,
        ],
    )
    def rotary_gather(pos_hbm, table_hbm, cos_hbm, sin_hbm,
                      pos_v, rows_v, cbuf, sbuf, sem_g, sem_wc, sem_ws):
        wid = lax.axis_index("s") * nc + lax.axis_index("c")
        row0 = wid * rows_w
        pltpu.sync_copy(pos_hbm.at[pl.ds(row0, rows_w)], pos_v)

        def issue_gather(j):
            return pltpu.async_copy(
                table_hbm.at[pos_v.at[j]], rows_v.at[j % 2], sem_g)

        def split(b):
            @pl.loop(0, CHUNK // 2)
            def _(k):
                r0 = 2 * k
                r1 = r0 + 1
                for c in range(HALF // 16):
                    lo = pl.ds(c * 16, 16)
                    hi = pl.ds(HALF + c * 16, 16)
                    cbuf[b, k, lo] = rows_v[b, r0, lo]
                    cbuf[b, k, hi] = rows_v[b, r1, lo]
                    sbuf[b, k, lo] = rows_v[b, r0, hi]
                    sbuf[b, k, hi] = rows_v[b, r1, hi]

        def issue_writes(j):
            b = j % 2
            base = (row0 + j) * (CHUNK // 2)
            dst = pl.ds(base, CHUNK // 2)
            return (
                pltpu.async_copy(cbuf.at[b], cos_hbm.at[dst], sem_wc),
                pltpu.async_copy(sbuf.at[b], sin_hbm.at[dst], sem_ws),
            )

        g_desc = {0: issue_gather(0)}
        w_desc = {}
        for j in range(rows_w):
            g_desc[j].wait()
            if j + 1 < rows_w:
                g_desc[j + 1] = issue_gather(j + 1)
            if j - 2 >= 0:
                for d in w_desc[j - 2]:
                    d.wait()
            split(j % 2)
            w_desc[j] = issue_writes(j)
        for j in (rows_w - 2, rows_w - 1):
            for d in w_desc[j]:
                d.wait()

    return rotary_gather


def kernel(positions, cos_sin_cache):
    pos = positions.astype(jnp.int32).reshape(N_ROWS, CHUNK)
    table = cos_sin_cache.reshape(SEQ, HEAD_SIZE)
    cos_p, sin_p = _build_sc_kernel()(pos, table)
    cos = cos_p.reshape(BATCH, 1, SEQ, HALF)
    sin = sin_p.reshape(BATCH, 1, SEQ, HALF)
    return (cos, sin)
